# baseline (speedup 1.0000x reference)
import functools

import numpy as np
import jax
import jax.numpy as jnp
from jax import lax
from jax.experimental import pallas as pl
from jax.experimental.pallas import tpu as pltpu
from jax.experimental.pallas import tpu_sc as plsc

B = 16
TOTAL = 4096
L_PAD = 512
MAX_SEL = 76
SEL_PROB = 0.15
VOCAB = 30522
MASK_TOKEN = 103
MASK_RATE, RAND_RATE = 0.8, 0.1

NCHUNK = L_PAD // 16
MPAD = 80


def _threefry2x32(k0, k1, x0, x1):
    def rotl(x, d):
        return ((x << np.uint32(d)) | (x >> np.uint32(32 - d))).astype(np.uint32)

    ks0, ks1 = np.uint32(k0), np.uint32(k1)
    ks2 = np.uint32(np.uint32(0x1BD11BDA) ^ ks0 ^ ks1)
    rotations = ((13, 15, 26, 6), (17, 29, 16, 24))
    x0 = (x0 + ks0).astype(np.uint32)
    x1 = (x1 + ks1).astype(np.uint32)
    ks = (ks0, ks1, ks2)
    for d in range(5):
        for r in rotations[d % 2]:
            x0 = (x0 + x1).astype(np.uint32)
            x1 = rotl(x1, r)
            x1 = (x1 ^ x0).astype(np.uint32)
        x0 = (x0 + ks[(d + 1) % 3]).astype(np.uint32)
        x1 = (x1 + ks[(d + 2) % 3] + np.uint32(d + 1)).astype(np.uint32)
    return x0, x1


def _random_bits(keypair, n):
    b1, b2 = _threefry2x32(keypair[0], keypair[1],
                           np.zeros(n, np.uint32), np.arange(n, dtype=np.uint32))
    return b1 ^ b2


def _split_key(keypair, num):
    b1, b2 = _threefry2x32(keypair[0], keypair[1],
                           np.zeros(num, np.uint32), np.arange(num, dtype=np.uint32))
    return [(b1[i], b2[i]) for i in range(num)]


def _np_uniform(keypair, n):
    bits = _random_bits(keypair, n)
    float_bits = (bits >> np.uint32(9)) | np.uint32(0x3F800000)
    return float_bits.view(np.float32) - np.float32(1.0)


def _np_randint(keypair, n, span):
    k1, k2 = _split_key(keypair, 2)
    higher, lower = _random_bits(k1, n), _random_bits(k2, n)
    span = np.uint32(span)
    mult = np.uint32((int(2 ** 16 % int(span)) ** 2) % int(span))
    return (((higher % span) * mult + (lower % span)) % span).astype(np.int32)


def _make_packed_constants() -> np.ndarray:
    n = B * L_PAD
    ksel, kchoice, krand = _split_key((np.uint32(0), np.uint32(42)), 3)
    u = _np_uniform(ksel, n).reshape(B, L_PAD)
    rr = _np_uniform(kchoice, n).reshape(B, L_PAD)
    rand_tok = _np_randint(krand, n, VOCAB).reshape(B, L_PAD)
    selmask = (u < np.float32(SEL_PROB)).astype(np.int32)
    keep = (rr >= np.float32(MASK_RATE + RAND_RATE)).astype(np.int32)
    prechosen = np.where(rr < np.float32(MASK_RATE),
                         np.int32(MASK_TOKEN), rand_tok).astype(np.int32)
    return prechosen | (keep << 16) | (selmask << 17)


_PACKED = _make_packed_constants()


def _build_sc_seq_mask():
    mesh = plsc.VectorSubcoreMesh(core_axis_name="c", subcore_axis_name="s")
    return functools.partial(
        pl.kernel,
        out_type=[
            jax.ShapeDtypeStruct((B, L_PAD), jnp.int32),
            jax.ShapeDtypeStruct((B, MPAD), jnp.int32),
            jax.ShapeDtypeStruct((B, MPAD), jnp.int32),
            jax.ShapeDtypeStruct((B, L_PAD), jnp.int32),
        ],
        mesh=mesh,
        compiler_params=pltpu.CompilerParams(needs_layout_passes=False),
        scratch_types=[
            pltpu.VMEM((TOTAL,), jnp.int32),
            pltpu.VMEM((16,), jnp.int32),
            pltpu.VMEM((L_PAD,), jnp.int32),
            pltpu.VMEM((L_PAD,), jnp.int32),
            pltpu.VMEM((L_PAD,), jnp.int32),
            pltpu.VMEM((MPAD,), jnp.int32),
            pltpu.VMEM((MPAD,), jnp.int32),
            pltpu.SemaphoreType.DMA,
        ],
    )(_sc_seq_mask_body)


def _sc_seq_mask_body(flat_hbm, cu_hbm, packed_hbm,
                 ids_hbm, mpos_hbm, mval_hbm, tt_hbm,
                 flat_v, cu_v, packed_v, ids_v, tt_v, mpos_v, mval_v, sem):
    c = lax.axis_index("c")
    s = lax.axis_index("s")
    wid = s * 2 + c

    @pl.when(wid < B)
    def _():
        r = wid
        h1 = pltpu.async_copy(flat_hbm, flat_v, sem)
        h2 = pltpu.async_copy(cu_hbm.at[pl.ds(0, 16)], cu_v, sem)
        h3 = pltpu.async_copy(packed_hbm.at[r], packed_v, sem)

        iota = lax.iota(jnp.int32, 16)
        zero = jnp.zeros((16,), jnp.int32)
        for k in range(MPAD // 16):
            mpos_v[pl.ds(16 * k, 16)] = zero
            mval_v[pl.ds(16 * k, 16)] = zero

        h1.wait()
        h2.wait()
        h3.wait()

        rfull = jnp.full((16,), r, jnp.int32)
        start = plsc.load_gather(cu_v, [rfull])
        end = jnp.where(rfull == B - 1, TOTAL,
                        plsc.load_gather(cu_v, [jnp.minimum(rfull + 1, B - 1)]))
        length = end - start
        clamp = jnp.maximum(length - 1, 0)

        def body(j, carry):
            off = pl.multiple_of(j * 16, 16)
            pos = j * 16 + iota
            idx = start + jnp.minimum(pos, clamp)
            tok = plsc.load_gather(flat_v, [idx])
            valid = pos < length
            padded = jnp.where(valid, tok, 0)
            w = packed_v[pl.ds(off, 16)]
            sel = valid & (padded >= 4) & ((w >> 17) != 0)
            csum = carry + plsc.cumsum(sel.astype(jnp.int32))
            sel_f = sel & (csum <= MAX_SEL)
            chosen = jnp.where((w & (1 << 16)) != 0, padded, w & 0xFFFF)
            ids_v[pl.ds(off, 16)] = jnp.where(sel_f, chosen, padded)
            tt_v[pl.ds(off, 16)] = valid.astype(jnp.int32)
            plsc.store_scatter(mpos_v, [csum - 1], pos, mask=sel_f)
            plsc.store_scatter(mval_v, [csum - 1], padded, mask=sel_f)
            return jnp.full((16,), jnp.max(csum), jnp.int32)

        lax.fori_loop(0, NCHUNK, body, zero)

        o1 = pltpu.async_copy(ids_v, ids_hbm.at[r], sem)
        o2 = pltpu.async_copy(tt_v, tt_hbm.at[r], sem)
        o3 = pltpu.async_copy(mpos_v, mpos_hbm.at[r], sem)
        o4 = pltpu.async_copy(mval_v, mval_hbm.at[r], sem)
        o1.wait()
        o2.wait()
        o3.wait()
        o4.wait()


_SC_SEQ_MASK = None


def kernel(flat_tokens, cu_seqlens):
    global _SC_SEQ_MASK
    if _SC_SEQ_MASK is None:
        _SC_SEQ_MASK = _build_sc_seq_mask()
    packed = jnp.asarray(_PACKED)
    ids, mpos, mval, tt = _SC_SEQ_MASK(
        flat_tokens.astype(jnp.int32), cu_seqlens.astype(jnp.int32), packed)
    return ids, mpos[:, :MAX_SEL], mval[:, :MAX_SEL], tt


# speedup vs baseline: 1.0040x; 1.0040x over previous
import functools

import numpy as np
import jax
import jax.numpy as jnp
from jax import lax
from jax.experimental import pallas as pl
from jax.experimental.pallas import tpu as pltpu
from jax.experimental.pallas import tpu_sc as plsc

B = 16
TOTAL = 4096
L_PAD = 512
MAX_SEL = 76
SEL_PROB = 0.15
VOCAB = 30522
MASK_TOKEN = 103
MASK_RATE, RAND_RATE = 0.8, 0.1

NCHUNK = L_PAD // 16
MPAD = 80


def _threefry2x32(k0, k1, x0, x1):
    def rotl(x, d):
        return ((x << np.uint32(d)) | (x >> np.uint32(32 - d))).astype(np.uint32)

    ks0, ks1 = np.uint32(k0), np.uint32(k1)
    ks2 = np.uint32(np.uint32(0x1BD11BDA) ^ ks0 ^ ks1)
    rotations = ((13, 15, 26, 6), (17, 29, 16, 24))
    x0 = (x0 + ks0).astype(np.uint32)
    x1 = (x1 + ks1).astype(np.uint32)
    ks = (ks0, ks1, ks2)
    for d in range(5):
        for r in rotations[d % 2]:
            x0 = (x0 + x1).astype(np.uint32)
            x1 = rotl(x1, r)
            x1 = (x1 ^ x0).astype(np.uint32)
        x0 = (x0 + ks[(d + 1) % 3]).astype(np.uint32)
        x1 = (x1 + ks[(d + 2) % 3] + np.uint32(d + 1)).astype(np.uint32)
    return x0, x1


def _random_bits(keypair, n):
    b1, b2 = _threefry2x32(keypair[0], keypair[1],
                           np.zeros(n, np.uint32), np.arange(n, dtype=np.uint32))
    return b1 ^ b2


def _split_key(keypair, num):
    b1, b2 = _threefry2x32(keypair[0], keypair[1],
                           np.zeros(num, np.uint32), np.arange(num, dtype=np.uint32))
    return [(b1[i], b2[i]) for i in range(num)]


def _np_uniform(keypair, n):
    bits = _random_bits(keypair, n)
    float_bits = (bits >> np.uint32(9)) | np.uint32(0x3F800000)
    return float_bits.view(np.float32) - np.float32(1.0)


def _np_randint(keypair, n, span):
    k1, k2 = _split_key(keypair, 2)
    higher, lower = _random_bits(k1, n), _random_bits(k2, n)
    span = np.uint32(span)
    mult = np.uint32((int(2 ** 16 % int(span)) ** 2) % int(span))
    return (((higher % span) * mult + (lower % span)) % span).astype(np.int32)


def _make_packed_constants() -> np.ndarray:
    n = B * L_PAD
    ksel, kchoice, krand = _split_key((np.uint32(0), np.uint32(42)), 3)
    u = _np_uniform(ksel, n).reshape(B, L_PAD)
    rr = _np_uniform(kchoice, n).reshape(B, L_PAD)
    rand_tok = _np_randint(krand, n, VOCAB).reshape(B, L_PAD)
    selmask = (u < np.float32(SEL_PROB)).astype(np.int32)
    keep = (rr >= np.float32(MASK_RATE + RAND_RATE)).astype(np.int32)
    prechosen = np.where(rr < np.float32(MASK_RATE),
                         np.int32(MASK_TOKEN), rand_tok).astype(np.int32)
    return (prechosen | (keep << 16) | (selmask << 17)).reshape(-1)


_PACKED = _make_packed_constants()


def _build_sc_seq_mask():
    mesh = plsc.VectorSubcoreMesh(core_axis_name="c", subcore_axis_name="s")
    return functools.partial(
        pl.kernel,
        out_type=[
            jax.ShapeDtypeStruct((B, L_PAD), jnp.int32),
            jax.ShapeDtypeStruct((B, MPAD), jnp.int32),
            jax.ShapeDtypeStruct((B, MPAD), jnp.int32),
            jax.ShapeDtypeStruct((B, L_PAD), jnp.int32),
        ],
        mesh=mesh,
        compiler_params=pltpu.CompilerParams(needs_layout_passes=False),
        scratch_types=[
            pltpu.VMEM((TOTAL,), jnp.int32),
            pltpu.VMEM((16,), jnp.int32),
            pltpu.VMEM((L_PAD,), jnp.int32),
            pltpu.VMEM((L_PAD,), jnp.int32),
            pltpu.VMEM((L_PAD,), jnp.int32),
            pltpu.VMEM((MPAD,), jnp.int32),
            pltpu.VMEM((MPAD,), jnp.int32),
            pltpu.SemaphoreType.DMA,
        ],
    )(_sc_seq_mask_body)


def _sc_seq_mask_body(flat_hbm, cu_hbm, packed_hbm,
                 ids_hbm, mpos_hbm, mval_hbm, tt_hbm,
                 flat_v, cu_v, packed_v, ids_v, tt_v, mpos_v, mval_v, sem):
    c = lax.axis_index("c")
    s = lax.axis_index("s")
    wid = s * 2 + c

    @pl.when(wid < B)
    def _():
        r = wid
        h1 = pltpu.async_copy(flat_hbm, flat_v, sem)
        h2 = pltpu.async_copy(cu_hbm.at[pl.ds(0, 16)], cu_v, sem)
        h3 = pltpu.async_copy(packed_hbm.at[pl.ds(r * L_PAD, L_PAD)],
                              packed_v, sem)

        iota = lax.iota(jnp.int32, 16)
        zero = jnp.zeros((16,), jnp.int32)
        for k in range(MPAD // 16):
            mpos_v[pl.ds(16 * k, 16)] = zero
            mval_v[pl.ds(16 * k, 16)] = zero

        h1.wait()
        h2.wait()
        h3.wait()

        rfull = jnp.full((16,), r, jnp.int32)
        start = plsc.load_gather(cu_v, [rfull])
        end = jnp.where(rfull == B - 1, TOTAL,
                        plsc.load_gather(cu_v, [jnp.minimum(rfull + 1, B - 1)]))
        length = end - start
        clamp = jnp.maximum(length - 1, 0)

        def body(j, carry):
            off = pl.multiple_of(j * 16, 16)
            pos = j * 16 + iota
            idx = start + jnp.minimum(pos, clamp)
            tok = plsc.load_gather(flat_v, [idx])
            valid = pos < length
            padded = jnp.where(valid, tok, 0)
            w = packed_v[pl.ds(off, 16)]
            sel = valid & (padded >= 4) & ((w >> 17) != 0)
            csum = carry + plsc.cumsum(sel.astype(jnp.int32))
            sel_f = sel & (csum <= MAX_SEL)
            chosen = jnp.where((w & (1 << 16)) != 0, padded, w & 0xFFFF)
            ids_v[pl.ds(off, 16)] = jnp.where(sel_f, chosen, padded)
            tt_v[pl.ds(off, 16)] = valid.astype(jnp.int32)
            plsc.store_scatter(mpos_v, [csum - 1], pos, mask=sel_f)
            plsc.store_scatter(mval_v, [csum - 1], padded, mask=sel_f)
            return jnp.full((16,), jnp.max(csum), jnp.int32)

        lax.fori_loop(0, NCHUNK, body, zero)

        o1 = pltpu.async_copy(ids_v, ids_hbm.at[r], sem)
        o2 = pltpu.async_copy(tt_v, tt_hbm.at[r], sem)
        o3 = pltpu.async_copy(mpos_v, mpos_hbm.at[r], sem)
        o4 = pltpu.async_copy(mval_v, mval_hbm.at[r], sem)
        o1.wait()
        o2.wait()
        o3.wait()
        o4.wait()


_SC_SEQ_MASK = None


def kernel(flat_tokens, cu_seqlens):
    global _SC_SEQ_MASK
    if _SC_SEQ_MASK is None:
        _SC_SEQ_MASK = _build_sc_seq_mask()
    packed = jnp.asarray(_PACKED)
    ids, mpos, mval, tt = _SC_SEQ_MASK(
        flat_tokens.astype(jnp.int32), cu_seqlens.astype(jnp.int32), packed)
    return ids, mpos[:, :MAX_SEL], mval[:, :MAX_SEL], tt
